# Initial kernel scaffold; baseline (speedup 1.0000x reference)
import jax
import jax.numpy as jnp
from jax.experimental import pallas as pl


def kernel(xyz, feats, params):
    raise NotImplementedError("write your pallas kernel here")


# speedup vs baseline: 2.2991x; 2.2991x over previous
import numpy as np
import jax
import jax.numpy as jnp
from jax import lax
from jax.experimental import pallas as pl
from jax.experimental.pallas import tpu as pltpu

_RADIUS2 = 0.25
_K = 32
_H = 8
_EPS = 1e-5
_L = 128


def _fps_body(x_ref, y_ref, z_ref, idx_ref, cx_ref, cy_ref, cz_ref):
    B, S, L = x_ref.shape
    M = idx_ref.shape[1]
    x = x_ref[...]
    y = y_ref[...]
    z = z_ref[...]
    pos = (lax.broadcasted_iota(jnp.int32, (B, S, L), 1) * L
           + lax.broadcasted_iota(jnp.int32, (B, S, L), 2))

    def dist(px, py, pz):
        dx = x - px
        dy = y - py
        dz = z - pz
        return (dx * dx + dy * dy) + dz * dz

    def red(a, op):
        return op(op(a, axis=2, keepdims=True), axis=1, keepdims=True)

    p0x = x[:, 0:1, 0:1]
    p0y = y[:, 0:1, 0:1]
    p0z = z[:, 0:1, 0:1]
    idx_ref[:, 0:1, :] = jnp.zeros((B, 1, 1), jnp.int32)
    cx_ref[:, 0:1, :] = p0x
    cy_ref[:, 0:1, :] = p0y
    cz_ref[:, 0:1, :] = p0z
    mind0 = dist(p0x, p0y, p0z)

    def body(i, mind):
        mx = red(mind, jnp.max)
        sel = jnp.where(mind == mx, pos, jnp.int32(2 ** 30))
        far = red(sel, jnp.min)
        oh = pos == far
        fx = red(jnp.where(oh, x, 0.0), jnp.sum)
        fy = red(jnp.where(oh, y, 0.0), jnp.sum)
        fz = red(jnp.where(oh, z, 0.0), jnp.sum)
        idx_ref[:, pl.ds(i, 1), :] = far
        cx_ref[:, pl.ds(i, 1), :] = fx
        cy_ref[:, pl.ds(i, 1), :] = fy
        cz_ref[:, pl.ds(i, 1), :] = fz
        return jnp.minimum(mind, dist(fx, fy, fz))

    lax.fori_loop(1, M, body, mind0)


def _ballq_body(cx_ref, cy_ref, cz_ref, x_ref, y_ref, z_ref, raw_ref,
                rank_ref):
    R = cx_ref.shape[0]
    _, S, L = x_ref.shape
    N = S * L
    cx = cx_ref[...]
    cy = cy_ref[...]
    cz = cz_ref[...]
    c2 = (cx * cx + cy * cy) + cz * cz
    bf = lambda a: a.astype(jnp.bfloat16).astype(jnp.float32)
    cxb = bf(cx)
    cyb = bf(cy)
    czb = bf(cz)
    l1 = lax.broadcasted_iota(jnp.int32, (L, L), 0)
    l2 = lax.broadcasted_iota(jnp.int32, (L, L), 1)
    tri = (l1 <= l2).astype(jnp.float32)
    xv = x_ref[0]
    yv = y_ref[0]
    zv = z_ref[0]
    carry = jnp.zeros((R, 1), jnp.float32)
    for t in range(S):
        xt = xv[t:t + 1, :]
        yt = yv[t:t + 1, :]
        zt = zv[t:t + 1, :]
        dott = (cxb * bf(xt) + cyb * bf(yt)) + czb * bf(zt)
        x2t = (xt * xt + yt * yt) + zt * zt
        d2 = (c2 + x2t) - 2.0 * dott
        valid = (d2 < _RADIUS2).astype(jnp.float32)
        rank = jnp.dot(valid, tri,
                       preferred_element_type=jnp.float32) + carry
        carry = rank[:, L - 1:L]
        rank_ref[:, t * L:(t + 1) * L] = rank
    for k in range(_K):
        acc = None
        kf = jnp.float32(k)
        for t in range(S):
            m = (rank_ref[:, t * L:(t + 1) * L] <= kf).astype(jnp.float32)
            acc = m if acc is None else acc + m
        nk = jnp.sum(acc, axis=1, keepdims=True).astype(jnp.int32)
        raw_ref[:, k:k + 1] = jnp.where(nk < N, nk, -1)


def _ln(xv, g, b):
    mu = jnp.mean(xv, axis=-1, keepdims=True)
    xc = xv - mu
    var = jnp.mean(xc * xc, axis=-1, keepdims=True)
    return xc * lax.rsqrt(var + _EPS) * g + b


def _dense_body(gf_ref, rawsub_ref, ctrsub_ref, rawl_ref,
                inw_ref, inb_ref, ing_ref, inbb_ref,
                wq_ref, bq_ref, wk_ref, bk_ref, wv_ref, bv_ref,
                wo_ref, bo_ref, ln1g_ref, ln1b_ref,
                f1w_ref, f1b_ref, f2w_ref, f2b_ref, ln2g_ref, ln2b_ref,
                og_ref, ob_ref, out_ref):
    T = rawsub_ref.shape[0]
    N, F = gf_ref.shape[1], gf_ref.shape[2]
    C = inw_ref.shape[1]
    dh = C // _H

    raw = rawsub_ref[...]
    ctr = ctrsub_ref[...]
    nidx = jnp.where(raw < 0, ctr, raw)
    padl = rawl_ref[0]
    feats = gf_ref[0]
    colbase = lax.broadcasted_iota(jnp.int32, (T, _L), 1)
    acc = jnp.zeros((T, F), jnp.float32)
    for t in range(N // _L):
        oh = (colbase + (t * _L) == nidx).astype(jnp.float32)
        acc = acc + jnp.dot(oh, feats[t * _L:(t + 1) * _L, :],
                            preferred_element_type=jnp.float32)

    h0 = jnp.dot(acc, inw_ref[...],
                 preferred_element_type=jnp.float32) + inb_ref[...]
    h0 = jnp.maximum(_ln(h0, ing_ref[...], inbb_ref[...]), 0.0)

    pad = padl < 0
    gi = lax.broadcasted_iota(jnp.int32, (T, T), 0) // _K
    gj = lax.broadcasted_iota(jnp.int32, (T, T), 1) // _K
    keymask = (gi == gj) & jnp.logical_not(pad)

    scale = jnp.float32(1.0 / np.sqrt(dh))
    attn = jnp.zeros((T, C), jnp.float32)
    for h in range(_H):
        qh = jnp.dot(h0, wq_ref[h],
                     preferred_element_type=jnp.float32) + bq_ref[h]
        kh = jnp.dot(h0, wk_ref[h],
                     preferred_element_type=jnp.float32) + bk_ref[h]
        vh = jnp.dot(h0, wv_ref[h],
                     preferred_element_type=jnp.float32) + bv_ref[h]
        s = lax.dot_general(qh, kh, (((1,), (1,)), ((), ())),
                            preferred_element_type=jnp.float32) * scale
        s = jnp.where(keymask, s, jnp.float32(-1e9))
        mx = jnp.max(s, axis=1, keepdims=True)
        p = jnp.exp(s - mx)
        sm = p / jnp.sum(p, axis=1, keepdims=True)
        ov = jnp.dot(sm, vh, preferred_element_type=jnp.float32)
        attn = attn + jnp.dot(ov, wo_ref[h],
                              preferred_element_type=jnp.float32)
    attn = attn + bo_ref[...]

    x1 = _ln(h0 + attn, ln1g_ref[...], ln1b_ref[...])
    f = jnp.maximum(jnp.dot(x1, f1w_ref[...],
                            preferred_element_type=jnp.float32)
                    + f1b_ref[...], 0.0)
    f = jnp.dot(f, f2w_ref[...],
                preferred_element_type=jnp.float32) + f2b_ref[...]
    x2 = _ln(x1 + f, ln2g_ref[...], ln2b_ref[...])

    pools = [jnp.max(x2[g * _K:(g + 1) * _K, :], axis=0, keepdims=True)
             for g in range(T // _K)]
    pooled = jnp.concatenate(pools, axis=0)
    out_ref[...] = _ln(pooled, og_ref[...], ob_ref[...])


def kernel(xyz, feats, params):
    B, N, _ = xyz.shape
    M = int(round(N * 0.25))
    S = N // _L
    F = feats.shape[2]
    C = params['in_w'].shape[1]
    FF = params['ff1_w'].shape[1]
    dh = C // _H

    xs = xyz[..., 0].reshape(B, S, _L)
    ys = xyz[..., 1].reshape(B, S, _L)
    zs = xyz[..., 2].reshape(B, S, _L)

    o = jax.ShapeDtypeStruct
    idx3, cx3, cy3, cz3 = pl.pallas_call(
        _fps_body,
        out_shape=[o((B, M, 1), jnp.int32), o((B, M, 1), jnp.float32),
                   o((B, M, 1), jnp.float32), o((B, M, 1), jnp.float32)],
    )(xs, ys, zs)
    ctr_idx = idx3[..., 0]
    ctr_xyz = jnp.concatenate([cx3, cy3, cz3], axis=-1)

    RB = 128
    nblk = (B * M) // RB
    blk_per_b = M // RB
    cxr = cx3.reshape(B * M, 1)
    cyr = cy3.reshape(B * M, 1)
    czr = cz3.reshape(B * M, 1)
    raw = pl.pallas_call(
        _ballq_body,
        grid=(nblk,),
        in_specs=[
            pl.BlockSpec((RB, 1), lambda g: (g, 0)),
            pl.BlockSpec((RB, 1), lambda g: (g, 0)),
            pl.BlockSpec((RB, 1), lambda g: (g, 0)),
            pl.BlockSpec((1, S, _L), lambda g: (g // blk_per_b, 0, 0)),
            pl.BlockSpec((1, S, _L), lambda g: (g // blk_per_b, 0, 0)),
            pl.BlockSpec((1, S, _L), lambda g: (g // blk_per_b, 0, 0)),
        ],
        out_specs=pl.BlockSpec((RB, _K), lambda g: (g, 0)),
        out_shape=o((B * M, _K), jnp.int32),
        scratch_shapes=[pltpu.VMEM((RB, N), jnp.float32)],
    )(cxr, cyr, czr, xs, ys, zs)

    TOT = B * M * _K
    T = 256
    G = TOT // T
    chunks_per_b = G // B
    raw_sub = raw.reshape(TOT, 1)
    ctr_sub = jnp.broadcast_to(ctr_idx[:, :, None],
                               (B, M, _K)).reshape(TOT, 1)
    rawl = raw.reshape(G, 1, T)

    p = params
    wq = p['Wq'].reshape(C, _H, dh).transpose(1, 0, 2)
    wk = p['Wk'].reshape(C, _H, dh).transpose(1, 0, 2)
    wv = p['Wv'].reshape(C, _H, dh).transpose(1, 0, 2)
    wo = p['Wo'].reshape(_H, dh, C)
    bq = p['bq'].reshape(_H, 1, dh)
    bk = p['bk'].reshape(_H, 1, dh)
    bv = p['bv'].reshape(_H, 1, dh)
    r1 = lambda a: a.reshape(1, -1)

    full = lambda shape: pl.BlockSpec(shape, lambda g: tuple(0 for _ in shape))
    pooled = pl.pallas_call(
        _dense_body,
        grid=(G,),
        in_specs=[
            pl.BlockSpec((1, N, F), lambda g: (g // chunks_per_b, 0, 0)),
            pl.BlockSpec((T, 1), lambda g: (g, 0)),
            pl.BlockSpec((T, 1), lambda g: (g, 0)),
            pl.BlockSpec((1, 1, T), lambda g: (g, 0, 0)),
            full((F, C)), full((1, C)), full((1, C)), full((1, C)),
            full((_H, C, dh)), full((_H, 1, dh)),
            full((_H, C, dh)), full((_H, 1, dh)),
            full((_H, C, dh)), full((_H, 1, dh)),
            full((_H, dh, C)), full((1, C)), full((1, C)), full((1, C)),
            full((C, FF)), full((1, FF)), full((FF, C)), full((1, C)),
            full((1, C)), full((1, C)), full((1, C)), full((1, C)),
        ],
        out_specs=pl.BlockSpec((T // _K, C), lambda g: (g, 0)),
        out_shape=o((B * M, C), jnp.float32),
    )(feats, raw_sub, ctr_sub, rawl,
      p['in_w'], r1(p['in_b']), r1(p['in_ln_g']), r1(p['in_ln_b']),
      wq, bq, wk, bk, wv, bv, wo, r1(p['bo']),
      r1(p['ln1_g']), r1(p['ln1_b']),
      p['ff1_w'], r1(p['ff1_b']), p['ff2_w'], r1(p['ff2_b']),
      r1(p['ln2_g']), r1(p['ln2_b']),
      r1(p['out_ln_g']), r1(p['out_ln_b']))

    return ctr_xyz, pooled.reshape(B, M, C)


# speedup vs baseline: 2.5401x; 1.1048x over previous
import functools

import numpy as np
import jax
import jax.numpy as jnp
from jax import lax
from jax.experimental import pallas as pl
from jax.experimental.pallas import tpu as pltpu
from jax.experimental.pallas import tpu_sc as plsc

_RADIUS2 = 0.25
_K = 32
_H = 8
_EPS = 1e-5
_L = 128


def _fps_body(x_ref, y_ref, z_ref, idx_ref, cx_ref, cy_ref, cz_ref):
    B, S, L = x_ref.shape
    M = idx_ref.shape[1]
    x = x_ref[...]
    y = y_ref[...]
    z = z_ref[...]
    pos = (lax.broadcasted_iota(jnp.int32, (B, S, L), 1) * L
           + lax.broadcasted_iota(jnp.int32, (B, S, L), 2))

    def dist(px, py, pz):
        dx = x - px
        dy = y - py
        dz = z - pz
        return (dx * dx + dy * dy) + dz * dz

    def red(a, op):
        return op(op(a, axis=2, keepdims=True), axis=1, keepdims=True)

    p0x = x[:, 0:1, 0:1]
    p0y = y[:, 0:1, 0:1]
    p0z = z[:, 0:1, 0:1]
    idx_ref[:, 0:1, :] = jnp.zeros((B, 1, 1), jnp.int32)
    cx_ref[:, 0:1, :] = p0x
    cy_ref[:, 0:1, :] = p0y
    cz_ref[:, 0:1, :] = p0z
    mind0 = dist(p0x, p0y, p0z)

    def body(i, mind):
        mx = red(mind, jnp.max)
        sel = jnp.where(mind == mx, pos, jnp.int32(2 ** 30))
        far = red(sel, jnp.min)
        oh = pos == far
        fx = red(jnp.where(oh, x, 0.0), jnp.sum)
        fy = red(jnp.where(oh, y, 0.0), jnp.sum)
        fz = red(jnp.where(oh, z, 0.0), jnp.sum)
        idx_ref[:, pl.ds(i, 1), :] = far
        cx_ref[:, pl.ds(i, 1), :] = fx
        cy_ref[:, pl.ds(i, 1), :] = fy
        cz_ref[:, pl.ds(i, 1), :] = fz
        return jnp.minimum(mind, dist(fx, fy, fz))

    lax.fori_loop(1, M, body, mind0)


def _ballq_body(cx_ref, cy_ref, cz_ref, x_ref, y_ref, z_ref, raw_ref,
                rank_ref):
    R = cx_ref.shape[0]
    _, S, L = x_ref.shape
    N = S * L
    cx = cx_ref[...]
    cy = cy_ref[...]
    cz = cz_ref[...]
    c2 = (cx * cx + cy * cy) + cz * cz
    bf = lambda a: a.astype(jnp.bfloat16).astype(jnp.float32)
    cxb = bf(cx)
    cyb = bf(cy)
    czb = bf(cz)
    l1 = lax.broadcasted_iota(jnp.int32, (L, L), 0)
    l2 = lax.broadcasted_iota(jnp.int32, (L, L), 1)
    tri = (l1 <= l2).astype(jnp.float32)
    xv = x_ref[0]
    yv = y_ref[0]
    zv = z_ref[0]
    carry = jnp.zeros((R, 1), jnp.float32)
    for t in range(S):
        xt = xv[t:t + 1, :]
        yt = yv[t:t + 1, :]
        zt = zv[t:t + 1, :]
        dott = (cxb * bf(xt) + cyb * bf(yt)) + czb * bf(zt)
        x2t = (xt * xt + yt * yt) + zt * zt
        d2 = (c2 + x2t) - 2.0 * dott
        valid = (d2 < _RADIUS2).astype(jnp.float32)
        rank = jnp.dot(valid, tri,
                       preferred_element_type=jnp.float32) + carry
        carry = rank[:, L - 1:L]
        rank_ref[:, t * L:(t + 1) * L] = rank
    for k in range(_K):
        acc = None
        kf = jnp.float32(k)
        for t in range(S):
            m = (rank_ref[:, t * L:(t + 1) * L] <= kf).astype(jnp.float32)
            acc = m if acc is None else acc + m
        nk = jnp.sum(acc, axis=1, keepdims=True).astype(jnp.int32)
        raw_ref[:, k:k + 1] = jnp.where(nk < N, nk, -1)


def _make_sc_gather(B, N, M, F):
    TOT = B * M * _K
    info = plsc.get_sparse_core_info()
    NW = info.num_cores * info.num_subcores
    NC = info.num_cores
    rows_w = TOT // NW
    CH = 128
    nch = rows_w // CH
    grp_w = rows_w // _K
    wpb = NW // B
    mesh = plsc.VectorSubcoreMesh(core_axis_name="c", subcore_axis_name="s")

    @functools.partial(
        pl.kernel, mesh=mesh,
        out_type=jax.ShapeDtypeStruct((TOT, F), jnp.float32),
        scratch_types=[
            pltpu.VMEM((grp_w,), jnp.int32),
            pltpu.VMEM((CH,), jnp.int32),
            pltpu.VMEM((CH, F), jnp.float32),
            pltpu.SemaphoreType.DMA,
        ],
    )
    def k(feats_hbm, raw_hbm, ctr_hbm, out_hbm, ctr_v, idx_v, rows_v, sem):
        wid = lax.axis_index("s") * NC + lax.axis_index("c")
        base = wid * rows_w
        pltpu.sync_copy(ctr_hbm.at[pl.ds(wid * grp_w, grp_w)], ctr_v)
        boff = (wid // wpb) * N
        for ch in range(nch):
            cb = base + ch * CH
            pltpu.sync_copy(raw_hbm.at[pl.ds(cb, CH)], idx_v)
            for j in range(CH // 16):
                r = idx_v[pl.ds(j * 16, 16)]
                g = (ch * CH + j * 16) // _K
                c = ctr_v[pl.ds((g // 16) * 16, 16)][g % 16]
                idx_v[pl.ds(j * 16, 16)] = jnp.where(r < 0, c, r) + boff
            pltpu.async_copy(feats_hbm.at[idx_v], rows_v, sem).wait()
            pltpu.sync_copy(rows_v, out_hbm.at[pl.ds(cb, CH)])

    return k


def _ln(xv, g, b):
    mu = jnp.mean(xv, axis=-1, keepdims=True)
    xc = xv - mu
    var = jnp.mean(xc * xc, axis=-1, keepdims=True)
    return xc * lax.rsqrt(var + _EPS) * g + b


def _dense_body(gath_ref, rawl_ref,
                inw_ref, inb_ref, ing_ref, inbb_ref,
                wq_ref, bq_ref, wk_ref, bk_ref, wv_ref, bv_ref,
                wo_ref, bo_ref, ln1g_ref, ln1b_ref,
                f1w_ref, f1b_ref, f2w_ref, f2b_ref, ln2g_ref, ln2b_ref,
                og_ref, ob_ref, out_ref):
    T = gath_ref.shape[0]
    C = inw_ref.shape[1]
    dh = C // _H

    padl = rawl_ref[0]
    acc = gath_ref[...]

    h0 = jnp.dot(acc, inw_ref[...],
                 preferred_element_type=jnp.float32) + inb_ref[...]
    h0 = jnp.maximum(_ln(h0, ing_ref[...], inbb_ref[...]), 0.0)

    pad = padl < 0
    gi = lax.broadcasted_iota(jnp.int32, (T, T), 0) // _K
    gj = lax.broadcasted_iota(jnp.int32, (T, T), 1) // _K
    keymask = (gi == gj) & jnp.logical_not(pad)

    scale = jnp.float32(1.0 / np.sqrt(dh))
    attn = jnp.zeros((T, C), jnp.float32)
    for h in range(_H):
        qh = jnp.dot(h0, wq_ref[h],
                     preferred_element_type=jnp.float32) + bq_ref[h]
        kh = jnp.dot(h0, wk_ref[h],
                     preferred_element_type=jnp.float32) + bk_ref[h]
        vh = jnp.dot(h0, wv_ref[h],
                     preferred_element_type=jnp.float32) + bv_ref[h]
        s = lax.dot_general(qh, kh, (((1,), (1,)), ((), ())),
                            preferred_element_type=jnp.float32) * scale
        s = jnp.where(keymask, s, jnp.float32(-1e9))
        mx = jnp.max(s, axis=1, keepdims=True)
        p = jnp.exp(s - mx)
        sm = p / jnp.sum(p, axis=1, keepdims=True)
        ov = jnp.dot(sm, vh, preferred_element_type=jnp.float32)
        attn = attn + jnp.dot(ov, wo_ref[h],
                              preferred_element_type=jnp.float32)
    attn = attn + bo_ref[...]

    x1 = _ln(h0 + attn, ln1g_ref[...], ln1b_ref[...])
    f = jnp.maximum(jnp.dot(x1, f1w_ref[...],
                            preferred_element_type=jnp.float32)
                    + f1b_ref[...], 0.0)
    f = jnp.dot(f, f2w_ref[...],
                preferred_element_type=jnp.float32) + f2b_ref[...]
    x2 = _ln(x1 + f, ln2g_ref[...], ln2b_ref[...])

    pools = [jnp.max(x2[g * _K:(g + 1) * _K, :], axis=0, keepdims=True)
             for g in range(T // _K)]
    pooled = jnp.concatenate(pools, axis=0)
    out_ref[...] = _ln(pooled, og_ref[...], ob_ref[...])


def kernel(xyz, feats, params):
    B, N, _ = xyz.shape
    M = int(round(N * 0.25))
    S = N // _L
    F = feats.shape[2]
    C = params['in_w'].shape[1]
    FF = params['ff1_w'].shape[1]
    dh = C // _H

    xs = xyz[..., 0].reshape(B, S, _L)
    ys = xyz[..., 1].reshape(B, S, _L)
    zs = xyz[..., 2].reshape(B, S, _L)

    o = jax.ShapeDtypeStruct
    idx3, cx3, cy3, cz3 = pl.pallas_call(
        _fps_body,
        out_shape=[o((B, M, 1), jnp.int32), o((B, M, 1), jnp.float32),
                   o((B, M, 1), jnp.float32), o((B, M, 1), jnp.float32)],
    )(xs, ys, zs)
    ctr_idx = idx3[..., 0]
    ctr_xyz = jnp.concatenate([cx3, cy3, cz3], axis=-1)

    RB = 128
    nblk = (B * M) // RB
    blk_per_b = M // RB
    cxr = cx3.reshape(B * M, 1)
    cyr = cy3.reshape(B * M, 1)
    czr = cz3.reshape(B * M, 1)
    raw = pl.pallas_call(
        _ballq_body,
        grid=(nblk,),
        in_specs=[
            pl.BlockSpec((RB, 1), lambda g: (g, 0)),
            pl.BlockSpec((RB, 1), lambda g: (g, 0)),
            pl.BlockSpec((RB, 1), lambda g: (g, 0)),
            pl.BlockSpec((1, S, _L), lambda g: (g // blk_per_b, 0, 0)),
            pl.BlockSpec((1, S, _L), lambda g: (g // blk_per_b, 0, 0)),
            pl.BlockSpec((1, S, _L), lambda g: (g // blk_per_b, 0, 0)),
        ],
        out_specs=pl.BlockSpec((RB, _K), lambda g: (g, 0)),
        out_shape=o((B * M, _K), jnp.int32),
        scratch_shapes=[pltpu.VMEM((RB, N), jnp.float32)],
    )(cxr, cyr, czr, xs, ys, zs)

    TOT = B * M * _K
    FP = _L
    feats_pad = jnp.concatenate(
        [feats.reshape(B * N, F),
         jnp.zeros((B * N, FP - F), jnp.float32)], axis=1)
    gathered = _make_sc_gather(B, N, M, FP)(
        feats_pad, raw.reshape(TOT), ctr_idx.reshape(B * M))

    T = 256
    G = TOT // T
    rawl = raw.reshape(G, 1, T)

    p = params
    wq = p['Wq'].reshape(C, _H, dh).transpose(1, 0, 2)
    wk = p['Wk'].reshape(C, _H, dh).transpose(1, 0, 2)
    wv = p['Wv'].reshape(C, _H, dh).transpose(1, 0, 2)
    wo = p['Wo'].reshape(_H, dh, C)
    bq = p['bq'].reshape(_H, 1, dh)
    bk = p['bk'].reshape(_H, 1, dh)
    bv = p['bv'].reshape(_H, 1, dh)
    r1 = lambda a: a.reshape(1, -1)

    full = lambda shape: pl.BlockSpec(shape, lambda g: tuple(0 for _ in shape))
    pooled = pl.pallas_call(
        _dense_body,
        grid=(G,),
        in_specs=[
            pl.BlockSpec((T, FP), lambda g: (g, 0)),
            pl.BlockSpec((1, 1, T), lambda g: (g, 0, 0)),
            full((FP, C)), full((1, C)), full((1, C)), full((1, C)),
            full((_H, C, dh)), full((_H, 1, dh)),
            full((_H, C, dh)), full((_H, 1, dh)),
            full((_H, C, dh)), full((_H, 1, dh)),
            full((_H, dh, C)), full((1, C)), full((1, C)), full((1, C)),
            full((C, FF)), full((1, FF)), full((FF, C)), full((1, C)),
            full((1, C)), full((1, C)), full((1, C)), full((1, C)),
        ],
        out_specs=pl.BlockSpec((T // _K, C), lambda g: (g, 0)),
        out_shape=o((B * M, C), jnp.float32),
    )(gathered, rawl,
      jnp.concatenate([p['in_w'],
                       jnp.zeros((FP - F, C), jnp.float32)], axis=0),
      r1(p['in_b']), r1(p['in_ln_g']), r1(p['in_ln_b']),
      wq, bq, wk, bk, wv, bv, wo, r1(p['bo']),
      r1(p['ln1_g']), r1(p['ln1_b']),
      p['ff1_w'], r1(p['ff1_b']), p['ff2_w'], r1(p['ff2_b']),
      r1(p['ln2_g']), r1(p['ln2_b']),
      r1(p['out_ln_g']), r1(p['out_ln_b']))

    return ctr_xyz, pooled.reshape(B, M, C)


# speedup vs baseline: 3.1001x; 1.2205x over previous
import functools

import numpy as np
import jax
import jax.numpy as jnp
from jax import lax
from jax.experimental import pallas as pl
from jax.experimental.pallas import tpu as pltpu
from jax.experimental.pallas import tpu_sc as plsc

_RADIUS2 = 0.25
_K = 32
_H = 8
_EPS = 1e-5
_L = 128


def _fps_body(x_ref, y_ref, z_ref, idx_ref, cx_ref, cy_ref, cz_ref):
    B, S, L = x_ref.shape
    M = idx_ref.shape[1]
    x = x_ref[...]
    y = y_ref[...]
    z = z_ref[...]
    pos = (lax.broadcasted_iota(jnp.int32, (B, S, L), 1) * L
           + lax.broadcasted_iota(jnp.int32, (B, S, L), 2))

    def dist(px, py, pz):
        dx = x - px
        dy = y - py
        dz = z - pz
        return (dx * dx + dy * dy) + dz * dz

    def red(a, op):
        return op(op(a, axis=2, keepdims=True), axis=1, keepdims=True)

    p0x = x[:, 0:1, 0:1]
    p0y = y[:, 0:1, 0:1]
    p0z = z[:, 0:1, 0:1]
    idx_ref[:, 0:1, :] = jnp.zeros((B, 1, 1), jnp.int32)
    cx_ref[:, 0:1, :] = p0x
    cy_ref[:, 0:1, :] = p0y
    cz_ref[:, 0:1, :] = p0z
    mind0 = dist(p0x, p0y, p0z)

    def body(i, mind):
        mx = red(mind, jnp.max)
        sel = jnp.where(mind == mx, pos, jnp.int32(2 ** 30))
        far = red(sel, jnp.min)
        oh = pos == far
        fx = red(jnp.where(oh, x, 0.0), jnp.sum)
        fy = red(jnp.where(oh, y, 0.0), jnp.sum)
        fz = red(jnp.where(oh, z, 0.0), jnp.sum)
        idx_ref[:, pl.ds(i, 1), :] = far
        cx_ref[:, pl.ds(i, 1), :] = fx
        cy_ref[:, pl.ds(i, 1), :] = fy
        cz_ref[:, pl.ds(i, 1), :] = fz
        return jnp.minimum(mind, dist(fx, fy, fz))

    lax.fori_loop(1, M, body, mind0)


def _ballq_body(cx_ref, cy_ref, cz_ref, x_ref, y_ref, z_ref, raw_ref,
                rank_ref):
    R = cx_ref.shape[0]
    _, S, L = x_ref.shape
    N = S * L
    cx = cx_ref[...]
    cy = cy_ref[...]
    cz = cz_ref[...]
    c2 = (cx * cx + cy * cy) + cz * cz
    bf = lambda a: a.astype(jnp.bfloat16).astype(jnp.float32)
    cxb = bf(cx)
    cyb = bf(cy)
    czb = bf(cz)
    l1 = lax.broadcasted_iota(jnp.int32, (L, L), 0)
    l2 = lax.broadcasted_iota(jnp.int32, (L, L), 1)
    tri = (l1 <= l2).astype(jnp.float32)
    xv = x_ref[0]
    yv = y_ref[0]
    zv = z_ref[0]
    carry = jnp.zeros((R, 1), jnp.float32)
    for t in range(S):
        xt = xv[t:t + 1, :]
        yt = yv[t:t + 1, :]
        zt = zv[t:t + 1, :]
        dott = (cxb * bf(xt) + cyb * bf(yt)) + czb * bf(zt)
        x2t = (xt * xt + yt * yt) + zt * zt
        d2 = (c2 + x2t) - 2.0 * dott
        valid = (d2 < _RADIUS2).astype(jnp.float32)
        rank = jnp.dot(valid, tri,
                       preferred_element_type=jnp.float32) + carry
        carry = rank[:, L - 1:L]
        rank_ref[:, t * L:(t + 1) * L] = rank
    for k in range(_K):
        acc = None
        kf = jnp.float32(k)
        for t in range(S):
            m = (rank_ref[:, t * L:(t + 1) * L] <= kf).astype(jnp.float32)
            acc = m if acc is None else acc + m
        nk = jnp.sum(acc, axis=1, keepdims=True).astype(jnp.int32)
        raw_ref[:, k:k + 1] = jnp.where(nk < N, nk, -1)


def _make_sc_gather(B, N, M, F):
    TOT = B * M * _K
    info = plsc.get_sparse_core_info()
    NW = info.num_cores * info.num_subcores
    NC = info.num_cores
    rows_w = TOT // NW
    CH = 128
    nch = rows_w // CH
    grp_w = rows_w // _K
    wpb = NW // B
    mesh = plsc.VectorSubcoreMesh(core_axis_name="c", subcore_axis_name="s")

    @functools.partial(
        pl.kernel, mesh=mesh,
        out_type=jax.ShapeDtypeStruct((TOT, F), jnp.float32),
        scratch_types=[
            pltpu.VMEM((grp_w,), jnp.int32),
            pltpu.VMEM((CH,), jnp.int32),
            pltpu.VMEM((CH, F), jnp.float32),
            pltpu.SemaphoreType.DMA,
        ],
    )
    def k(feats_hbm, raw_hbm, ctr_hbm, out_hbm, ctr_v, idx_v, rows_v, sem):
        wid = lax.axis_index("s") * NC + lax.axis_index("c")
        base = wid * rows_w
        pltpu.sync_copy(ctr_hbm.at[pl.ds(wid * grp_w, grp_w)], ctr_v)
        boff = (wid // wpb) * N
        for ch in range(nch):
            cb = base + ch * CH
            pltpu.sync_copy(raw_hbm.at[pl.ds(cb, CH)], idx_v)
            for j in range(CH // 16):
                r = idx_v[pl.ds(j * 16, 16)]
                g = (ch * CH + j * 16) // _K
                c = ctr_v[pl.ds((g // 16) * 16, 16)][g % 16]
                idx_v[pl.ds(j * 16, 16)] = jnp.where(r < 0, c, r) + boff
            pltpu.async_copy(feats_hbm.at[idx_v], rows_v, sem).wait()
            pltpu.sync_copy(rows_v, out_hbm.at[pl.ds(cb, CH)])

    return k


def _ln(xv, g, b):
    mu = jnp.mean(xv, axis=-1, keepdims=True)
    xc = xv - mu
    var = jnp.mean(xc * xc, axis=-1, keepdims=True)
    return xc * lax.rsqrt(var + _EPS) * g + b


def _dense_body(gath_ref, rawl_ref,
                inw_ref, inb_ref, ing_ref, inbb_ref,
                wq_ref, bq_ref, wk_ref, bk_ref, wv_ref, bv_ref,
                wo_ref, bo_ref, ln1g_ref, ln1b_ref,
                f1w_ref, f1b_ref, f2w_ref, f2b_ref, ln2g_ref, ln2b_ref,
                og_ref, ob_ref, out_ref):
    T = gath_ref.shape[0]
    C = inw_ref.shape[1]
    dh = C // _H

    rawt = rawl_ref[0]
    acc = gath_ref[...]

    h0 = jnp.dot(acc, inw_ref[...],
                 preferred_element_type=jnp.float32) + inb_ref[...]
    h0 = jnp.maximum(_ln(h0, ing_ref[...], inbb_ref[...]), 0.0)

    gi = lax.broadcasted_iota(jnp.int32, (C, C), 0) // dh
    gj = lax.broadcasted_iota(jnp.int32, (C, C), 1) // dh
    hblk = gi == gj
    hblkf = hblk.astype(jnp.float32)

    scale = jnp.float32(1.0 / np.sqrt(dh))
    qa = jnp.dot(h0, wq_ref[...],
                 preferred_element_type=jnp.float32) + bq_ref[...]
    ka = jnp.dot(h0, wk_ref[...],
                 preferred_element_type=jnp.float32) + bk_ref[...]
    va = jnp.dot(h0, wv_ref[...],
                 preferred_element_type=jnp.float32) + bv_ref[...]
    kt = jnp.transpose(ka)
    attn_rows = []
    for g in range(T // _K):
        qg = qa[g * _K:(g + 1) * _K, :]
        ktg = kt[:, g * _K:(g + 1) * _K]
        wg = jnp.where(hblk, jnp.concatenate([ktg] * _H, axis=1), 0.0)
        s2 = jnp.dot(qg, wg, preferred_element_type=jnp.float32) * scale
        padt = rawt[g:g + 1, :] < 0
        m2 = jnp.max(s2, axis=1, keepdims=True)
        p2 = jnp.where(padt, 0.0, jnp.exp(s2 - m2))
        den = jnp.dot(p2, hblkf, preferred_element_type=jnp.float32)
        sm2 = p2 / den
        vg = va[g * _K:(g + 1) * _K, :]
        vbd = jnp.where(hblk, jnp.concatenate([vg] * _H, axis=0), 0.0)
        attn_rows.append(jnp.dot(sm2, vbd,
                                 preferred_element_type=jnp.float32))
    attn = jnp.concatenate(attn_rows, axis=0)
    attn = jnp.dot(attn, wo_ref[...],
                   preferred_element_type=jnp.float32) + bo_ref[...]

    x1 = _ln(h0 + attn, ln1g_ref[...], ln1b_ref[...])
    f = jnp.maximum(jnp.dot(x1, f1w_ref[...],
                            preferred_element_type=jnp.float32)
                    + f1b_ref[...], 0.0)
    f = jnp.dot(f, f2w_ref[...],
                preferred_element_type=jnp.float32) + f2b_ref[...]
    x2 = _ln(x1 + f, ln2g_ref[...], ln2b_ref[...])

    pools = [jnp.max(x2[g * _K:(g + 1) * _K, :], axis=0, keepdims=True)
             for g in range(T // _K)]
    pooled = jnp.concatenate(pools, axis=0)
    out_ref[...] = _ln(pooled, og_ref[...], ob_ref[...])


def kernel(xyz, feats, params):
    B, N, _ = xyz.shape
    M = int(round(N * 0.25))
    S = N // _L
    F = feats.shape[2]
    C = params['in_w'].shape[1]
    FF = params['ff1_w'].shape[1]
    dh = C // _H

    xs = xyz[..., 0].reshape(B, S, _L)
    ys = xyz[..., 1].reshape(B, S, _L)
    zs = xyz[..., 2].reshape(B, S, _L)

    o = jax.ShapeDtypeStruct
    idx3, cx3, cy3, cz3 = pl.pallas_call(
        _fps_body,
        out_shape=[o((B, M, 1), jnp.int32), o((B, M, 1), jnp.float32),
                   o((B, M, 1), jnp.float32), o((B, M, 1), jnp.float32)],
    )(xs, ys, zs)
    ctr_idx = idx3[..., 0]
    ctr_xyz = jnp.concatenate([cx3, cy3, cz3], axis=-1)

    RB = 128
    nblk = (B * M) // RB
    blk_per_b = M // RB
    cxr = cx3.reshape(B * M, 1)
    cyr = cy3.reshape(B * M, 1)
    czr = cz3.reshape(B * M, 1)
    raw = pl.pallas_call(
        _ballq_body,
        grid=(nblk,),
        in_specs=[
            pl.BlockSpec((RB, 1), lambda g: (g, 0)),
            pl.BlockSpec((RB, 1), lambda g: (g, 0)),
            pl.BlockSpec((RB, 1), lambda g: (g, 0)),
            pl.BlockSpec((1, S, _L), lambda g: (g // blk_per_b, 0, 0)),
            pl.BlockSpec((1, S, _L), lambda g: (g // blk_per_b, 0, 0)),
            pl.BlockSpec((1, S, _L), lambda g: (g // blk_per_b, 0, 0)),
        ],
        out_specs=pl.BlockSpec((RB, _K), lambda g: (g, 0)),
        out_shape=o((B * M, _K), jnp.int32),
        scratch_shapes=[pltpu.VMEM((RB, N), jnp.float32)],
    )(cxr, cyr, czr, xs, ys, zs)

    TOT = B * M * _K
    FP = _L
    feats_pad = jnp.concatenate(
        [feats.reshape(B * N, F),
         jnp.zeros((B * N, FP - F), jnp.float32)], axis=1)
    gathered = _make_sc_gather(B, N, M, FP)(
        feats_pad, raw.reshape(TOT), ctr_idx.reshape(B * M))

    T = 256
    G = TOT // T
    GPC = T // _K
    rawl = jnp.tile(raw.reshape(B * M, 1, _K),
                    (1, _H, 1)).reshape(G, GPC, C)

    p = params
    r1 = lambda a: a.reshape(1, -1)

    full = lambda shape: pl.BlockSpec(shape, lambda g: tuple(0 for _ in shape))
    pooled = pl.pallas_call(
        _dense_body,
        grid=(G,),
        in_specs=[
            pl.BlockSpec((T, FP), lambda g: (g, 0)),
            pl.BlockSpec((1, GPC, C), lambda g: (g, 0, 0)),
            full((FP, C)), full((1, C)), full((1, C)), full((1, C)),
            full((C, C)), full((1, C)),
            full((C, C)), full((1, C)),
            full((C, C)), full((1, C)),
            full((C, C)), full((1, C)), full((1, C)), full((1, C)),
            full((C, FF)), full((1, FF)), full((FF, C)), full((1, C)),
            full((1, C)), full((1, C)), full((1, C)), full((1, C)),
        ],
        out_specs=pl.BlockSpec((T // _K, C), lambda g: (g, 0)),
        out_shape=o((B * M, C), jnp.float32),
    )(gathered, rawl,
      jnp.concatenate([p['in_w'],
                       jnp.zeros((FP - F, C), jnp.float32)], axis=0),
      r1(p['in_b']), r1(p['in_ln_g']), r1(p['in_ln_b']),
      p['Wq'], r1(p['bq']), p['Wk'], r1(p['bk']), p['Wv'], r1(p['bv']),
      p['Wo'], r1(p['bo']),
      r1(p['ln1_g']), r1(p['ln1_b']),
      p['ff1_w'], r1(p['ff1_b']), p['ff2_w'], r1(p['ff2_b']),
      r1(p['ln2_g']), r1(p['ln2_b']),
      r1(p['out_ln_g']), r1(p['out_ln_b']))

    return ctr_xyz, pooled.reshape(B, M, C)


# speedup vs baseline: 4.1169x; 1.3280x over previous
import functools

import numpy as np
import jax
import jax.numpy as jnp
from jax import lax
from jax.experimental import pallas as pl
from jax.experimental.pallas import tpu as pltpu
from jax.experimental.pallas import tpu_sc as plsc

_RADIUS2 = 0.25
_K = 32
_H = 8
_EPS = 1e-5
_L = 128


def _fps_body(x_ref, y_ref, z_ref, idx_ref, cx_ref, cy_ref, cz_ref):
    B, S, L = x_ref.shape
    M = idx_ref.shape[1]
    x = x_ref[...]
    y = y_ref[...]
    z = z_ref[...]
    pos = (lax.broadcasted_iota(jnp.int32, (B, S, L), 1) * L
           + lax.broadcasted_iota(jnp.int32, (B, S, L), 2))

    def dist(px, py, pz):
        dx = x - px
        dy = y - py
        dz = z - pz
        return (dx * dx + dy * dy) + dz * dz

    def red(a, op):
        return op(op(a, axis=2, keepdims=True), axis=1, keepdims=True)

    p0x = x[:, 0:1, 0:1]
    p0y = y[:, 0:1, 0:1]
    p0z = z[:, 0:1, 0:1]
    idx_ref[:, 0:1, :] = jnp.zeros((B, 1, 1), jnp.int32)
    cx_ref[:, 0:1, :] = p0x
    cy_ref[:, 0:1, :] = p0y
    cz_ref[:, 0:1, :] = p0z
    mind0 = dist(p0x, p0y, p0z)

    def body(i, mind):
        mx = red(mind, jnp.max)
        sel = jnp.where(mind == mx, pos, jnp.int32(2 ** 30))
        far = red(sel, jnp.min)
        oh = pos == far
        fx = red(jnp.where(oh, x, 0.0), jnp.sum)
        fy = red(jnp.where(oh, y, 0.0), jnp.sum)
        fz = red(jnp.where(oh, z, 0.0), jnp.sum)
        idx_ref[:, pl.ds(i, 1), :] = far
        cx_ref[:, pl.ds(i, 1), :] = fx
        cy_ref[:, pl.ds(i, 1), :] = fy
        cz_ref[:, pl.ds(i, 1), :] = fz
        return jnp.minimum(mind, dist(fx, fy, fz))

    lax.fori_loop(1, M, body, mind0)


def _ballq_body(cx_ref, cy_ref, cz_ref, x_ref, y_ref, z_ref, raw_ref,
                rank_ref):
    R = cx_ref.shape[0]
    _, S, L = x_ref.shape
    N = S * L
    cx = cx_ref[...]
    cy = cy_ref[...]
    cz = cz_ref[...]
    c2 = (cx * cx + cy * cy) + cz * cz
    bf = lambda a: a.astype(jnp.bfloat16).astype(jnp.float32)
    cxb = bf(cx)
    cyb = bf(cy)
    czb = bf(cz)
    l1 = lax.broadcasted_iota(jnp.int32, (L, L), 0)
    l2 = lax.broadcasted_iota(jnp.int32, (L, L), 1)
    tri = (l1 <= l2).astype(jnp.float32)
    xv = x_ref[0]
    yv = y_ref[0]
    zv = z_ref[0]
    carry = jnp.zeros((R, 1), jnp.float32)
    for t in range(S):
        xt = xv[t:t + 1, :]
        yt = yv[t:t + 1, :]
        zt = zv[t:t + 1, :]
        dott = (cxb * bf(xt) + cyb * bf(yt)) + czb * bf(zt)
        x2t = (xt * xt + yt * yt) + zt * zt
        d2 = (c2 + x2t) - 2.0 * dott
        valid = (d2 < _RADIUS2).astype(jnp.float32)
        rank = jnp.dot(valid, tri,
                       preferred_element_type=jnp.float32) + carry
        carry = rank[:, L - 1:L]
        rank_ref[:, t * L:(t + 1) * L] = rank
    for k in range(_K):
        acc = None
        kf = jnp.float32(k)
        for t in range(S):
            m = (rank_ref[:, t * L:(t + 1) * L] <= kf).astype(jnp.float32)
            acc = m if acc is None else acc + m
        nk = jnp.sum(acc, axis=1, keepdims=True).astype(jnp.int32)
        raw_ref[:, k:k + 1] = jnp.where(nk < N, nk, -1)


def _make_sc_gather(B, N, M, F):
    TOT = B * M * _K
    info = plsc.get_sparse_core_info()
    NW = info.num_cores * info.num_subcores
    NC = info.num_cores
    rows_w = TOT // NW
    CH = 128
    nch = rows_w // CH
    grp_w = rows_w // _K
    wpb = NW // B
    mesh = plsc.VectorSubcoreMesh(core_axis_name="c", subcore_axis_name="s")

    @functools.partial(
        pl.kernel, mesh=mesh,
        out_type=jax.ShapeDtypeStruct((TOT, F), jnp.float32),
        scratch_types=[
            pltpu.VMEM((grp_w,), jnp.int32),
            pltpu.VMEM((CH,), jnp.int32),
            pltpu.VMEM((CH, F), jnp.float32),
            pltpu.SemaphoreType.DMA,
        ],
    )
    def k(feats_hbm, raw_hbm, ctr_hbm, out_hbm, ctr_v, idx_v, rows_v, sem):
        wid = lax.axis_index("s") * NC + lax.axis_index("c")
        base = wid * rows_w
        pltpu.sync_copy(ctr_hbm.at[pl.ds(wid * grp_w, grp_w)], ctr_v)
        boff = (wid // wpb) * N
        for ch in range(nch):
            cb = base + ch * CH
            pltpu.sync_copy(raw_hbm.at[pl.ds(cb, CH)], idx_v)
            for j in range(CH // 16):
                r = idx_v[pl.ds(j * 16, 16)]
                g = (ch * CH + j * 16) // _K
                c = ctr_v[pl.ds((g // 16) * 16, 16)][g % 16]
                idx_v[pl.ds(j * 16, 16)] = jnp.where(r < 0, c, r) + boff
            pltpu.async_copy(feats_hbm.at[idx_v], rows_v, sem).wait()
            pltpu.sync_copy(rows_v, out_hbm.at[pl.ds(cb, CH)])

    return k


def _ln(xv, g, b):
    mu = jnp.mean(xv, axis=-1, keepdims=True)
    xc = xv - mu
    var = jnp.mean(xc * xc, axis=-1, keepdims=True)
    return xc * lax.rsqrt(var + _EPS) * g + b


def _dense_body(gath_ref, rawl_ref,
                inw_ref, inb_ref, ing_ref, inbb_ref,
                wq_ref, bq_ref, wk_ref, bk_ref, wv_ref, bv_ref,
                wo_ref, bo_ref, ln1g_ref, ln1b_ref,
                f1w_ref, f1b_ref, f2w_ref, f2b_ref, ln2g_ref, ln2b_ref,
                og_ref, ob_ref, out_ref):
    T = gath_ref.shape[0]
    C = inw_ref.shape[1]
    dh = C // _H

    rawt = rawl_ref[0]
    acc = gath_ref[...]

    h0 = jnp.dot(acc, inw_ref[...],
                 preferred_element_type=jnp.float32) + inb_ref[...]
    h0 = jnp.maximum(_ln(h0, ing_ref[...], inbb_ref[...]), 0.0)

    gi = lax.broadcasted_iota(jnp.int32, (C, C), 0) // dh
    gj = lax.broadcasted_iota(jnp.int32, (C, C), 1) // dh
    hblk = gi == gj
    hblkf = hblk.astype(jnp.float32)

    scale = jnp.float32(1.0 / np.sqrt(dh))
    qa = (jnp.dot(h0, wq_ref[...],
                  preferred_element_type=jnp.float32) + bq_ref[...]) * scale
    ka = jnp.dot(h0, wk_ref[...],
                 preferred_element_type=jnp.float32) + bk_ref[...]
    va = jnp.dot(h0, wv_ref[...],
                 preferred_element_type=jnp.float32) + bv_ref[...]
    kt = jnp.transpose(ka)
    srows = []
    for g in range(T // _K):
        qg = qa[g * _K:(g + 1) * _K, :]
        ktg = kt[:, g * _K:(g + 1) * _K]
        wg = jnp.where(hblk, jnp.concatenate([ktg] * _H, axis=1), 0.0)
        s2 = jnp.dot(qg, wg, preferred_element_type=jnp.float32)
        padt = rawt[g:g + 1, :] < 0
        srows.append(jnp.where(padt, jnp.float32(-1e9), s2))
    s_all = jnp.concatenate(srows, axis=0)
    m_all = jnp.max(s_all, axis=1, keepdims=True)
    p_all = jnp.exp(s_all - m_all)
    den = jnp.dot(p_all, hblkf, preferred_element_type=jnp.float32)
    sm_all = p_all / den
    arows = []
    for g in range(T // _K):
        vg = va[g * _K:(g + 1) * _K, :]
        vbd = jnp.where(hblk, jnp.concatenate([vg] * _H, axis=0), 0.0)
        arows.append(jnp.dot(sm_all[g * _K:(g + 1) * _K, :], vbd,
                             preferred_element_type=jnp.float32))
    attn = jnp.concatenate(arows, axis=0)
    attn = jnp.dot(attn, wo_ref[...],
                   preferred_element_type=jnp.float32) + bo_ref[...]

    x1 = _ln(h0 + attn, ln1g_ref[...], ln1b_ref[...])
    f = jnp.maximum(jnp.dot(x1, f1w_ref[...],
                            preferred_element_type=jnp.float32)
                    + f1b_ref[...], 0.0)
    f = jnp.dot(f, f2w_ref[...],
                preferred_element_type=jnp.float32) + f2b_ref[...]
    x2 = _ln(x1 + f, ln2g_ref[...], ln2b_ref[...])

    pools = [jnp.max(x2[g * _K:(g + 1) * _K, :], axis=0, keepdims=True)
             for g in range(T // _K)]
    pooled = jnp.concatenate(pools, axis=0)
    out_ref[...] = _ln(pooled, og_ref[...], ob_ref[...])


def kernel(xyz, feats, params):
    B, N, _ = xyz.shape
    M = int(round(N * 0.25))
    S = N // _L
    F = feats.shape[2]
    C = params['in_w'].shape[1]
    FF = params['ff1_w'].shape[1]
    dh = C // _H

    xs = xyz[..., 0].reshape(B, S, _L)
    ys = xyz[..., 1].reshape(B, S, _L)
    zs = xyz[..., 2].reshape(B, S, _L)

    o = jax.ShapeDtypeStruct
    idx3, cx3, cy3, cz3 = pl.pallas_call(
        _fps_body,
        out_shape=[o((B, M, 1), jnp.int32), o((B, M, 1), jnp.float32),
                   o((B, M, 1), jnp.float32), o((B, M, 1), jnp.float32)],
    )(xs, ys, zs)
    ctr_idx = idx3[..., 0]
    ctr_xyz = jnp.concatenate([cx3, cy3, cz3], axis=-1)

    RB = 128
    nblk = (B * M) // RB
    blk_per_b = M // RB
    cxr = cx3.reshape(B * M, 1)
    cyr = cy3.reshape(B * M, 1)
    czr = cz3.reshape(B * M, 1)
    raw = pl.pallas_call(
        _ballq_body,
        grid=(nblk,),
        in_specs=[
            pl.BlockSpec((RB, 1), lambda g: (g, 0)),
            pl.BlockSpec((RB, 1), lambda g: (g, 0)),
            pl.BlockSpec((RB, 1), lambda g: (g, 0)),
            pl.BlockSpec((1, S, _L), lambda g: (g // blk_per_b, 0, 0)),
            pl.BlockSpec((1, S, _L), lambda g: (g // blk_per_b, 0, 0)),
            pl.BlockSpec((1, S, _L), lambda g: (g // blk_per_b, 0, 0)),
        ],
        out_specs=pl.BlockSpec((RB, _K), lambda g: (g, 0)),
        out_shape=o((B * M, _K), jnp.int32),
        scratch_shapes=[pltpu.VMEM((RB, N), jnp.float32)],
    )(cxr, cyr, czr, xs, ys, zs)

    TOT = B * M * _K
    FP = _L
    feats_pad = jnp.concatenate(
        [feats.reshape(B * N, F),
         jnp.zeros((B * N, FP - F), jnp.float32)], axis=1)
    gathered = _make_sc_gather(B, N, M, FP)(
        feats_pad, raw.reshape(TOT), ctr_idx.reshape(B * M))

    T = 256
    G = TOT // T
    GPC = T // _K
    rawl = jnp.tile(raw.reshape(B * M, 1, _K),
                    (1, _H, 1)).reshape(G, GPC, C)

    p = params
    r1 = lambda a: a.reshape(1, -1)

    full = lambda shape: pl.BlockSpec(shape, lambda g: tuple(0 for _ in shape))
    pooled = pl.pallas_call(
        _dense_body,
        grid=(G,),
        in_specs=[
            pl.BlockSpec((T, FP), lambda g: (g, 0)),
            pl.BlockSpec((1, GPC, C), lambda g: (g, 0, 0)),
            full((FP, C)), full((1, C)), full((1, C)), full((1, C)),
            full((C, C)), full((1, C)),
            full((C, C)), full((1, C)),
            full((C, C)), full((1, C)),
            full((C, C)), full((1, C)), full((1, C)), full((1, C)),
            full((C, FF)), full((1, FF)), full((FF, C)), full((1, C)),
            full((1, C)), full((1, C)), full((1, C)), full((1, C)),
        ],
        out_specs=pl.BlockSpec((T // _K, C), lambda g: (g, 0)),
        out_shape=o((B * M, C), jnp.float32),
    )(gathered, rawl,
      jnp.concatenate([p['in_w'],
                       jnp.zeros((FP - F, C), jnp.float32)], axis=0),
      r1(p['in_b']), r1(p['in_ln_g']), r1(p['in_ln_b']),
      p['Wq'], r1(p['bq']), p['Wk'], r1(p['bk']), p['Wv'], r1(p['bv']),
      p['Wo'], r1(p['bo']),
      r1(p['ln1_g']), r1(p['ln1_b']),
      p['ff1_w'], r1(p['ff1_b']), p['ff2_w'], r1(p['ff2_b']),
      r1(p['ln2_g']), r1(p['ln2_b']),
      r1(p['out_ln_g']), r1(p['out_ln_b']))

    return ctr_xyz, pooled.reshape(B, M, C)


# speedup vs baseline: 4.2400x; 1.0299x over previous
import functools

import numpy as np
import jax
import jax.numpy as jnp
from jax import lax
from jax.experimental import pallas as pl
from jax.experimental.pallas import tpu as pltpu
from jax.experimental.pallas import tpu_sc as plsc

_RADIUS2 = 0.25
_K = 32
_H = 8
_EPS = 1e-5
_L = 128


def _fps_body(x_ref, y_ref, z_ref, idx_ref, cx_ref, cy_ref, cz_ref):
    B, S, L = x_ref.shape
    M = idx_ref.shape[1]
    x = x_ref[...]
    y = y_ref[...]
    z = z_ref[...]
    pos = (lax.broadcasted_iota(jnp.int32, (B, S, L), 1) * L
           + lax.broadcasted_iota(jnp.int32, (B, S, L), 2))

    def dist(px, py, pz):
        dx = x - px
        dy = y - py
        dz = z - pz
        return (dx * dx + dy * dy) + dz * dz

    def red(a, op):
        return op(op(a, axis=2, keepdims=True), axis=1, keepdims=True)

    p0x = x[:, 0:1, 0:1]
    p0y = y[:, 0:1, 0:1]
    p0z = z[:, 0:1, 0:1]
    idx_ref[:, 0:1, :] = jnp.zeros((B, 1, 1), jnp.int32)
    cx_ref[:, 0:1, :] = p0x
    cy_ref[:, 0:1, :] = p0y
    cz_ref[:, 0:1, :] = p0z
    mind0 = dist(p0x, p0y, p0z)

    def body(i, mind):
        mx = red(mind, jnp.max)
        sel = jnp.where(mind == mx, pos, jnp.int32(2 ** 30))
        far = red(sel, jnp.min)
        oh = pos == far
        fx = red(jnp.where(oh, x, 0.0), jnp.sum)
        fy = red(jnp.where(oh, y, 0.0), jnp.sum)
        fz = red(jnp.where(oh, z, 0.0), jnp.sum)
        idx_ref[:, pl.ds(i, 1), :] = far
        cx_ref[:, pl.ds(i, 1), :] = fx
        cy_ref[:, pl.ds(i, 1), :] = fy
        cz_ref[:, pl.ds(i, 1), :] = fz
        return jnp.minimum(mind, dist(fx, fy, fz))

    lax.fori_loop(1, M, body, mind0)


def _ballq_body(cx_ref, cy_ref, cz_ref, x_ref, y_ref, z_ref, raw_ref,
                rank_ref):
    R = cx_ref.shape[0]
    _, S, L = x_ref.shape
    N = S * L
    cx = cx_ref[...]
    cy = cy_ref[...]
    cz = cz_ref[...]
    c2 = (cx * cx + cy * cy) + cz * cz
    bf = lambda a: a.astype(jnp.bfloat16).astype(jnp.float32)
    cxb = bf(cx)
    cyb = bf(cy)
    czb = bf(cz)
    l1 = lax.broadcasted_iota(jnp.int32, (L, L), 0)
    l2 = lax.broadcasted_iota(jnp.int32, (L, L), 1)
    tri = (l1 <= l2).astype(jnp.float32)
    xv = x_ref[0]
    yv = y_ref[0]
    zv = z_ref[0]
    carry = jnp.zeros((R, 1), jnp.float32)
    for t in range(S):
        xt = xv[t:t + 1, :]
        yt = yv[t:t + 1, :]
        zt = zv[t:t + 1, :]
        dott = (cxb * bf(xt) + cyb * bf(yt)) + czb * bf(zt)
        x2t = (xt * xt + yt * yt) + zt * zt
        d2 = (c2 + x2t) - 2.0 * dott
        valid = (d2 < _RADIUS2).astype(jnp.float32)
        rank = jnp.dot(valid, tri,
                       preferred_element_type=jnp.float32) + carry
        carry = rank[:, L - 1:L]
        rank_ref[:, t * L:(t + 1) * L] = rank
    for k in range(_K):
        acc = None
        kf = jnp.float32(k)
        for t in range(S):
            m = (rank_ref[:, t * L:(t + 1) * L] <= kf).astype(jnp.float32)
            acc = m if acc is None else acc + m
        nk = jnp.sum(acc, axis=1, keepdims=True).astype(jnp.int32)
        raw_ref[:, k:k + 1] = jnp.where(nk < N, nk, -1)


def _make_sc_gather(B, N, M, F):
    TOT = B * M * _K
    info = plsc.get_sparse_core_info()
    NW = info.num_cores * info.num_subcores
    NC = info.num_cores
    rows_w = TOT // NW
    CH = 128
    nch = rows_w // CH
    grp_w = rows_w // _K
    wpb = NW // B
    mesh = plsc.VectorSubcoreMesh(core_axis_name="c", subcore_axis_name="s")

    @functools.partial(
        pl.kernel, mesh=mesh,
        out_type=jax.ShapeDtypeStruct((TOT, F), jnp.float32),
        scratch_types=[
            pltpu.VMEM((grp_w,), jnp.int32),
            pltpu.VMEM((nch, CH), jnp.int32),
            pltpu.VMEM((2, CH, F), jnp.float32),
            pltpu.SemaphoreType.DMA,
            pltpu.SemaphoreType.DMA,
        ],
    )
    def k(feats_hbm, raw_hbm, ctr_hbm, out_hbm, ctr_v, idx_v, rows_v,
          sem0, sem1):
        wid = lax.axis_index("s") * NC + lax.axis_index("c")
        base = wid * rows_w
        pltpu.sync_copy(ctr_hbm.at[pl.ds(wid * grp_w, grp_w)], ctr_v)
        pltpu.sync_copy(raw_hbm.at[pl.ds(wid * nch, nch)], idx_v)
        boff = (wid // wpb) * N
        for ch in range(nch):
            for j in range(CH // 16):
                r = idx_v[ch, pl.ds(j * 16, 16)]
                g = (ch * CH + j * 16) // _K
                c = ctr_v[pl.ds((g // 16) * 16, 16)][g % 16]
                idx_v[ch, pl.ds(j * 16, 16)] = jnp.where(r < 0, c, r) + boff
        sems = (sem0, sem1)
        cps = [None] * nch
        cps[0] = pltpu.async_copy(feats_hbm.at[idx_v.at[0]],
                                  rows_v.at[0], sems[0])
        for ch in range(nch):
            if ch + 1 < nch:
                cps[ch + 1] = pltpu.async_copy(
                    feats_hbm.at[idx_v.at[ch + 1]],
                    rows_v.at[(ch + 1) % 2], sems[(ch + 1) % 2])
            cps[ch].wait()
            pltpu.sync_copy(rows_v.at[ch % 2],
                            out_hbm.at[pl.ds(base + ch * CH, CH)])

    return k


def _ln(xv, g, b):
    mu = jnp.mean(xv, axis=-1, keepdims=True)
    xc = xv - mu
    var = jnp.mean(xc * xc, axis=-1, keepdims=True)
    return xc * lax.rsqrt(var + _EPS) * g + b


def _dense_body(gath_ref, rawl_ref,
                inw_ref, inb_ref, ing_ref, inbb_ref,
                wq_ref, bq_ref, wk_ref, bk_ref, wv_ref, bv_ref,
                wo_ref, bo_ref, ln1g_ref, ln1b_ref,
                f1w_ref, f1b_ref, f2w_ref, f2b_ref, ln2g_ref, ln2b_ref,
                og_ref, ob_ref, out_ref):
    T = gath_ref.shape[0]
    C = inw_ref.shape[1]
    dh = C // _H

    rawt = rawl_ref[0]
    acc = gath_ref[...]

    h0 = jnp.dot(acc, inw_ref[...],
                 preferred_element_type=jnp.float32) + inb_ref[...]
    h0 = jnp.maximum(_ln(h0, ing_ref[...], inbb_ref[...]), 0.0)

    gi = lax.broadcasted_iota(jnp.int32, (C, C), 0) // dh
    gj = lax.broadcasted_iota(jnp.int32, (C, C), 1) // dh
    hblk = gi == gj
    hblkf = hblk.astype(jnp.float32)

    scale = jnp.float32(1.0 / np.sqrt(dh))
    qa = (jnp.dot(h0, wq_ref[...],
                  preferred_element_type=jnp.float32) + bq_ref[...]) * scale
    ka = jnp.dot(h0, wk_ref[...],
                 preferred_element_type=jnp.float32) + bk_ref[...]
    va = jnp.dot(h0, wv_ref[...],
                 preferred_element_type=jnp.float32) + bv_ref[...]
    kt = jnp.transpose(ka)
    srows = []
    for g in range(T // _K):
        qg = qa[g * _K:(g + 1) * _K, :]
        ktg = kt[:, g * _K:(g + 1) * _K]
        wg = jnp.where(hblk, jnp.concatenate([ktg] * _H, axis=1), 0.0)
        s2 = jnp.dot(qg, wg, preferred_element_type=jnp.float32)
        padt = rawt[g:g + 1, :] < 0
        srows.append(jnp.where(padt, jnp.float32(-1e9), s2))
    s_all = jnp.concatenate(srows, axis=0)
    m_all = jnp.max(s_all, axis=1, keepdims=True)
    p_all = jnp.exp(s_all - m_all)
    den = jnp.dot(p_all, hblkf, preferred_element_type=jnp.float32)
    sm_all = p_all / den
    arows = []
    for g in range(T // _K):
        vg = va[g * _K:(g + 1) * _K, :]
        vbd = jnp.where(hblk, jnp.concatenate([vg] * _H, axis=0), 0.0)
        arows.append(jnp.dot(sm_all[g * _K:(g + 1) * _K, :], vbd,
                             preferred_element_type=jnp.float32))
    attn = jnp.concatenate(arows, axis=0)
    attn = jnp.dot(attn, wo_ref[...],
                   preferred_element_type=jnp.float32) + bo_ref[...]

    x1 = _ln(h0 + attn, ln1g_ref[...], ln1b_ref[...])
    f = jnp.maximum(jnp.dot(x1, f1w_ref[...],
                            preferred_element_type=jnp.float32)
                    + f1b_ref[...], 0.0)
    f = jnp.dot(f, f2w_ref[...],
                preferred_element_type=jnp.float32) + f2b_ref[...]
    x2 = _ln(x1 + f, ln2g_ref[...], ln2b_ref[...])

    pools = [jnp.max(x2[g * _K:(g + 1) * _K, :], axis=0, keepdims=True)
             for g in range(T // _K)]
    pooled = jnp.concatenate(pools, axis=0)
    out_ref[...] = _ln(pooled, og_ref[...], ob_ref[...])


def kernel(xyz, feats, params):
    B, N, _ = xyz.shape
    M = int(round(N * 0.25))
    S = N // _L
    F = feats.shape[2]
    C = params['in_w'].shape[1]
    FF = params['ff1_w'].shape[1]
    dh = C // _H

    xs = xyz[..., 0].reshape(B, S, _L)
    ys = xyz[..., 1].reshape(B, S, _L)
    zs = xyz[..., 2].reshape(B, S, _L)

    o = jax.ShapeDtypeStruct
    idx3, cx3, cy3, cz3 = pl.pallas_call(
        _fps_body,
        out_shape=[o((B, M, 1), jnp.int32), o((B, M, 1), jnp.float32),
                   o((B, M, 1), jnp.float32), o((B, M, 1), jnp.float32)],
    )(xs, ys, zs)
    ctr_idx = idx3[..., 0]
    ctr_xyz = jnp.concatenate([cx3, cy3, cz3], axis=-1)

    RB = 128
    nblk = (B * M) // RB
    blk_per_b = M // RB
    cxr = cx3.reshape(B * M, 1)
    cyr = cy3.reshape(B * M, 1)
    czr = cz3.reshape(B * M, 1)
    raw = pl.pallas_call(
        _ballq_body,
        grid=(nblk,),
        in_specs=[
            pl.BlockSpec((RB, 1), lambda g: (g, 0)),
            pl.BlockSpec((RB, 1), lambda g: (g, 0)),
            pl.BlockSpec((RB, 1), lambda g: (g, 0)),
            pl.BlockSpec((1, S, _L), lambda g: (g // blk_per_b, 0, 0)),
            pl.BlockSpec((1, S, _L), lambda g: (g // blk_per_b, 0, 0)),
            pl.BlockSpec((1, S, _L), lambda g: (g // blk_per_b, 0, 0)),
        ],
        out_specs=pl.BlockSpec((RB, _K), lambda g: (g, 0)),
        out_shape=o((B * M, _K), jnp.int32),
        scratch_shapes=[pltpu.VMEM((RB, N), jnp.float32)],
    )(cxr, cyr, czr, xs, ys, zs)

    TOT = B * M * _K
    FP = _L
    feats_pad = jnp.concatenate(
        [feats.reshape(B * N, F),
         jnp.zeros((B * N, FP - F), jnp.float32)], axis=1)
    gathered = _make_sc_gather(B, N, M, FP)(
        feats_pad, raw.reshape(TOT // _L, _L), ctr_idx.reshape(B * M))

    T = 256
    G = TOT // T
    GPC = T // _K
    rawl = jnp.tile(raw.reshape(B * M, 1, _K),
                    (1, _H, 1)).reshape(G, GPC, C)

    p = params
    r1 = lambda a: a.reshape(1, -1)

    full = lambda shape: pl.BlockSpec(shape, lambda g: tuple(0 for _ in shape))
    pooled = pl.pallas_call(
        _dense_body,
        grid=(G,),
        in_specs=[
            pl.BlockSpec((T, FP), lambda g: (g, 0)),
            pl.BlockSpec((1, GPC, C), lambda g: (g, 0, 0)),
            full((FP, C)), full((1, C)), full((1, C)), full((1, C)),
            full((C, C)), full((1, C)),
            full((C, C)), full((1, C)),
            full((C, C)), full((1, C)),
            full((C, C)), full((1, C)), full((1, C)), full((1, C)),
            full((C, FF)), full((1, FF)), full((FF, C)), full((1, C)),
            full((1, C)), full((1, C)), full((1, C)), full((1, C)),
        ],
        out_specs=pl.BlockSpec((T // _K, C), lambda g: (g, 0)),
        out_shape=o((B * M, C), jnp.float32),
    )(gathered, rawl,
      jnp.concatenate([p['in_w'],
                       jnp.zeros((FP - F, C), jnp.float32)], axis=0),
      r1(p['in_b']), r1(p['in_ln_g']), r1(p['in_ln_b']),
      p['Wq'], r1(p['bq']), p['Wk'], r1(p['bk']), p['Wv'], r1(p['bv']),
      p['Wo'], r1(p['bo']),
      r1(p['ln1_g']), r1(p['ln1_b']),
      p['ff1_w'], r1(p['ff1_b']), p['ff2_w'], r1(p['ff2_b']),
      r1(p['ln2_g']), r1(p['ln2_b']),
      r1(p['out_ln_g']), r1(p['out_ln_b']))

    return ctr_xyz, pooled.reshape(B, M, C)
